# baseline (speedup 1.0000x reference)
import functools

import jax
import jax.numpy as jnp
from jax import lax
from jax.experimental import pallas as pl
from jax.experimental.pallas import tpu as pltpu
from jax.experimental.pallas import tpu_sc as plsc

VOCAB = 1000000
NC, NS = 2, 16
NW = NC * NS
EMBED = 32
LANES = 16

BATCH, HIST = 4096, 200
TOTAL = BATCH * HIST
ROWPAD = 128
HB = 2
BB = 128
GRP = HB * BB
G = HIST // HB
EPI = 2 + 2 * ((G - 5) // 2)


def _make_gather():
  mesh = plsc.VectorSubcoreMesh(core_axis_name="c", subcore_axis_name="s")

  @functools.partial(
      pl.kernel,
      out_type=jax.ShapeDtypeStruct((HIST, EMBED // 8, NW, 8, BB),
                                    jnp.float32),
      mesh=mesh,
      scratch_types=[
          pltpu.VMEM((2, GRP), jnp.int32),
          pltpu.VMEM((2, GRP, ROWPAD), jnp.float32),
          pltpu.VMEM((HB * EMBED, 129), jnp.float32),
          pltpu.SemaphoreType.DMA,
          pltpu.SemaphoreType.DMA,
          pltpu.SemaphoreType.DMA,
      ],
      compiler_params=pltpu.CompilerParams(
          use_tc_tiling_on_sc=False, needs_layout_passes=False),
  )
  def gather_kernel(idx_hbm, table_hbm, out_hbm, idx_v, rows_v,
                    obuf, idx_sem, gat_sem, out_sem):
    wid = lax.axis_index("s") * NC + lax.axis_index("c")
    iota = lax.iota(jnp.int32, LANES)

    def idx_copy(g, slot):
      off = ((g // 4) * NW + wid) * (8 * BB) + (g % 4) * GRP
      return pltpu.make_async_copy(
          idx_hbm.at[pl.ds(off, GRP)], idx_v.at[slot], idx_sem)

    def gat(slot):
      return pltpu.make_async_copy(
          table_hbm.at[idx_v.at[slot]], rows_v.at[slot], gat_sem)

    def out_tile(g, mm, f):
      return pltpu.make_async_copy(
          obuf.at[pl.ds((mm * 4 + f) * 8, 8), pl.ds(0, BB)],
          out_hbm.at[g * HB + mm, f, wid], out_sem)

    def out_start(g):
      for mm in range(HB):
        for f in range(EMBED // 8):
          out_tile(g, mm, f).start()

    def out_wait(g):
      for mm in range(HB):
        for f in range(EMBED // 8):
          out_tile(g, mm, f).wait()

    def transpose(slot):
      @plsc.parallel_loop(0, HB, unroll=1)
      def _(m):
        r0 = m * EMBED + iota
        r1 = r0 + LANES

        @plsc.parallel_loop(0, BB, unroll=4)
        def _(b):
          row = m * BB + b
          colv = jnp.full((LANES,), b, jnp.int32)
          v0 = rows_v[slot, row, pl.ds(0, LANES)]
          v1 = rows_v[slot, row, pl.ds(LANES, LANES)]

          plsc.store_scatter(obuf, [r0, colv], v0)
          plsc.store_scatter(obuf, [r1, colv], v1)

    def step(g, s, have_f, have_il, have_ow):
      gat(s).wait()
      if have_f:
        idx_copy(g + 1, 1 - s).wait()
        gat(1 - s).start()
      if have_il:
        idx_copy(g + 2, s).start()
      if have_ow:
        out_wait(g - 1)
      transpose(s)
      out_start(g)

    idx_copy(0, 0).start()
    idx_copy(0, 0).wait()
    gat(0).start()
    idx_copy(1, 1).start()
    step(0, 0, True, True, False)
    step(1, 1, True, True, True)

    def pair(p, carry):
      for s in range(2):
        g = 2 * p + 2 + s
        step(g, s, True, True, True)
      return carry

    lax.fori_loop(0, (G - 5) // 2, pair, 0)

    for g in range(EPI, G):
      step(g, g % 2, g + 1 < G, g + 2 < G, True)
    out_wait(G - 1)

  return gather_kernel


_gather = _make_gather()


@jax.jit
def kernel(text, table):
  idx = (text.astype(jnp.int32)
         .reshape(NW, BB, HIST // 8, 8)
         .transpose(2, 0, 3, 1)
         .reshape(TOTAL))
  out5 = _gather(idx, jnp.pad(table, ((0, 0), (0, ROWPAD - EMBED))))
  return out5.transpose(2, 4, 0, 1, 3).reshape(BATCH, HIST, EMBED)


# speedup vs baseline: 1.2411x; 1.2411x over previous
import functools

import jax
import jax.numpy as jnp
from jax import lax
from jax.experimental import pallas as pl
from jax.experimental.pallas import tpu as pltpu
from jax.experimental.pallas import tpu_sc as plsc

VOCAB = 1000000
NC, NS = 2, 16
NW = NC * NS
EMBED = 32
LANES = 16

BATCH, HIST = 4096, 200
TOTAL = BATCH * HIST
GRP = 1024
HB = 8
BB = 128
G = HIST // HB


def _make_gather():
  mesh = plsc.VectorSubcoreMesh(core_axis_name="c", subcore_axis_name="s")

  @functools.partial(
      pl.kernel,
      out_type=jax.ShapeDtypeStruct((HIST, EMBED // 8, NW, HB, BB),
                                    jnp.float32),
      mesh=mesh,
      scratch_types=[
          pltpu.VMEM((2, GRP), jnp.int32),
          pltpu.VMEM((2, GRP, EMBED), jnp.float32),
          pltpu.VMEM((HB * EMBED, 129), jnp.float32),
          pltpu.SemaphoreType.DMA,
          pltpu.SemaphoreType.DMA,
          pltpu.SemaphoreType.DMA,
      ],
      compiler_params=pltpu.CompilerParams(
          use_tc_tiling_on_sc=False, needs_layout_passes=False),
  )
  def gather_kernel(idx_hbm, table_hbm, out_hbm, idx_v, rows_v,
                    obuf, idx_sem, gat_sem, out_sem):
    wid = lax.axis_index("s") * NC + lax.axis_index("c")
    iota = lax.iota(jnp.int32, LANES)

    def idx_copy(g, slot):
      return pltpu.make_async_copy(
          idx_hbm.at[pl.ds((g * NW + wid) * GRP, GRP)], idx_v.at[slot], idx_sem)

    def gat(slot):
      return pltpu.make_async_copy(
          table_hbm.at[idx_v.at[slot]], rows_v.at[slot], gat_sem)

    def out_tile(g, mm, f):
      return pltpu.make_async_copy(
          obuf.at[pl.ds((mm * 4 + f) * 8, 8), pl.ds(0, BB)],
          out_hbm.at[g * HB + mm, f, wid], out_sem)

    def out_start(g):
      for mm in range(HB):
        for f in range(EMBED // 8):
          out_tile(g, mm, f).start()

    def out_wait(g):
      for mm in range(HB):
        for f in range(EMBED // 8):
          out_tile(g, mm, f).wait()

    def transpose(slot):
      @plsc.parallel_loop(0, HB, unroll=1)
      def _(m):
        r0 = m * EMBED + iota
        r1 = r0 + LANES

        @plsc.parallel_loop(0, BB, unroll=4)
        def _(b):
          row = m * BB + b
          colv = jnp.full((LANES,), b, jnp.int32)
          v0 = rows_v[slot, row, pl.ds(0, LANES)]
          v1 = rows_v[slot, row, pl.ds(LANES, LANES)]
          plsc.store_scatter(obuf, [r0, colv], v0)
          plsc.store_scatter(obuf, [r1, colv], v1)

    def step(g, s, have_f, have_il, have_ow):
      gat(s).wait()
      if have_f:
        idx_copy(g + 1, 1 - s).wait()
        gat(1 - s).start()
      if have_il:
        idx_copy(g + 2, s).start()
      if have_ow:
        out_wait(g - 1)
      transpose(s)
      out_start(g)

    idx_copy(0, 0).start()
    idx_copy(0, 0).wait()
    gat(0).start()
    idx_copy(1, 1).start()
    step(0, 0, True, True, False)
    step(1, 1, True, True, True)

    def pair(p, carry):
      for s in range(2):
        g = 2 * p + 2 + s
        step(g, s, True, True, True)
      return carry

    lax.fori_loop(0, (G - 5) // 2, pair, 0)

    for g in range(G - 3, G):
      step(g, g % 2, g + 1 < G, g + 2 < G, True)
    out_wait(G - 1)

  return gather_kernel


_gather = _make_gather()


@jax.jit
def kernel(text, table):
  idx = (text.astype(jnp.int32)
         .reshape(NW, BB, HIST // HB, HB)
         .transpose(2, 0, 3, 1)
         .reshape(TOTAL))
  out5 = _gather(idx, table)
  return out5.transpose(2, 4, 0, 1, 3).reshape(BATCH, HIST, EMBED)
